# baseline (speedup 1.0000x reference)
import functools

import jax
import jax.numpy as jnp
from jax import lax
from jax.experimental import pallas as pl
from jax.experimental.pallas import tpu as pltpu
from jax.experimental.pallas import tpu_sc as plsc

_K = 4096
_LAMBDA = 0.5
_MOM = 0.5
_TEMP = 0.07
_M = 100000
_D = 256
_BS = 256
_BSG = 128

_CHUNK = 2048
_GRID = 49
_MPAD = _CHUNK * _GRID
_HALF = _CHUNK // 2
_PACKW = _MPAD // 2


def _rne_bf16(v):
    u = lax.bitcast_convert_type(v, jnp.int32)
    lsb = lax.shift_right_logical(u, 16) & 1
    return lax.shift_right_logical(u + jnp.int32(0x7FFF) + lsb, 16)


def _pack_bf16_pairs(x):
    lo = _rne_bf16(x[:, :_HALF])
    hi = _rne_bf16(x[:, _HALF:])
    return (hi << 16) | lo


def _rne_bf16(v):
    u = lax.bitcast_convert_type(v, jnp.int32)
    lsb = lax.shift_right_logical(u, 16) & 1
    return lax.shift_right_logical(u + jnp.int32(0x7FFF) + lsb, 16)


def _pack_bf16_pairs(x):
    lo = _rne_bf16(x[:, :_HALF])
    hi = _rne_bf16(x[:, _HALF:])
    return (hi << 16) | lo


def _k1_body(row_base, q_ref, bank_ref, qn_ref, copy_ref, l_ref):
    qf = q_ref[...]
    norm = jnp.sqrt(jnp.sum(qf * qf, axis=1, keepdims=True))
    qn = qf / (norm + 1e-12)

    @pl.when(pl.program_id(0) == 0)
    def _():
        qn_ref[...] = qn

    blk = bank_ref[...]
    copy_ref[...] = blk
    qh = qn[row_base:row_base + _BSG]
    l = lax.dot_general(qh, blk, (((1,), (1,)), ((), ())),
                        preferred_element_type=jnp.float32)
    l_ref[...] = _pack_bf16_pairs(l)


def _run_k1(q, bank, row_base):
    return pl.pallas_call(
        functools.partial(_k1_body, row_base),
        grid=(_GRID,),
        in_specs=[
            pl.BlockSpec((_BS, _D), lambda i: (0, 0)),
            pl.BlockSpec((_CHUNK, _D), lambda i: (i, 0)),
        ],
        out_specs=[
            pl.BlockSpec((_BS, _D), lambda i: (0, 0)),
            pl.BlockSpec((_CHUNK, _D), lambda i: (i, 0)),
            pl.BlockSpec((_BSG, _HALF), lambda i: (0, i)),
        ],
        out_shape=[
            jax.ShapeDtypeStruct((_BS, _D), jnp.float32),
            jax.ShapeDtypeStruct((_M, _D), jnp.float32),
            jax.ShapeDtypeStruct((_BSG, _HALF * _GRID), jnp.int32),
        ],
    )(q, bank)


_RPT = 4


def _newton_rsqrt(s):
    i = lax.bitcast_convert_type(s, jnp.int32)
    i = jnp.int32(0x5F3759DF) - lax.shift_right_arithmetic(i, 1)
    y = lax.bitcast_convert_type(i, jnp.float32)
    for _ in range(3):
        y = y * (1.5 - 0.5 * s * y * y)
    return y


def _decode_bf16(buf, g, inv_t):
    w = (lax.shift_right_logical(g, 1) & jnp.int32(-_HALF)) | (g & (_HALF - 1))
    word = plsc.load_gather(buf, [w])
    sh = lax.shift_right_logical(g, 6) & 16
    h = lax.shift_right_logical(word, sh) & jnp.int32(0xFFFF)
    f = plsc.bitcast(lax.shift_left(h, 16), jnp.float32)
    return f, jnp.exp(f * inv_t)


def _k2_body(row_base, L, bank, qn, idx2, neg2,
             upd_out, sums_out, pos_out,
             rowA, rowB, nidx_v, q_v, old_v, upd_v, idx_v, sums_v, pos_v,
             semA, semB, semC):
    inv_t = jnp.float32(1.0 / _TEMP)
    wid = lax.axis_index("s") * 2 + lax.axis_index("c")
    n0 = wid * _RPT
    pltpu.sync_copy(idx2.at[wid], idx_v)
    pltpu.sync_copy(qn.at[pl.ds(row_base + n0, _RPT)], q_v)
    pltpu.async_copy(bank.at[idx_v], old_v, semC).wait()
    lanes = lax.iota(jnp.int32, 16)

    def upd_body(j, _):
        ss = jnp.zeros((16,), jnp.float32)
        for c in range(_D // 16):
            u = (old_v[j, pl.ds(c * 16, 16)] * _MOM
                 + q_v[j, pl.ds(c * 16, 16)] * (1.0 - _MOM))
            ss = ss + u * u
        s2 = jnp.sum(ss)
        t = s2 * _newton_rsqrt(jnp.maximum(s2, jnp.float32(1e-30)))
        r = _newton_rsqrt(t + jnp.float32(1e-12))
        scale = r * r
        for c in range(_D // 16):
            u = (old_v[j, pl.ds(c * 16, 16)] * _MOM
                 + q_v[j, pl.ds(c * 16, 16)] * (1.0 - _MOM))
            upd_v[j, pl.ds(c * 16, 16)] = u * scale
        return 0

    lax.fori_loop(0, _RPT, upd_body, 0)
    pltpu.sync_copy(upd_v, upd_out.at[pl.ds(n0, _RPT)])

    def gather_one(buf, j, sums16, pos16):
        n = n0 + j
        pltpu.sync_copy(neg2.at[n], nidx_v)

        def body(k, accs):
            a0, a1, a2, a3 = accs
            i0 = nidx_v[pl.ds(k * 64, 16)]
            i1 = nidx_v[pl.ds(k * 64 + 16, 16)]
            i2 = nidx_v[pl.ds(k * 64 + 32, 16)]
            i3 = nidx_v[pl.ds(k * 64 + 48, 16)]
            a0 = a0 + _decode_bf16(buf, i0, inv_t)[1]
            a1 = a1 + _decode_bf16(buf, i1, inv_t)[1]
            a2 = a2 + _decode_bf16(buf, i2, inv_t)[1]
            a3 = a3 + _decode_bf16(buf, i3, inv_t)[1]
            return (a0, a1, a2, a3)

        z16 = jnp.zeros((16,), jnp.float32)
        a0, a1, a2, a3 = lax.fori_loop(0, _K // 64, body,
                                       (z16, z16, z16, z16))
        s_neg = jnp.sum((a0 + a1) + (a2 + a3))
        pidx = plsc.load_gather(idx_v, [jnp.full((16,), 0, jnp.int32) + j])
        pvec, pexp = _decode_bf16(buf, pidx, inv_t)
        p = jnp.sum(pvec) * jnp.float32(1.0 / 16.0)
        e_pos = jnp.sum(pexp) * jnp.float32(1.0 / 16.0)
        sums16 = jnp.where(lanes == j, s_neg + e_pos, sums16)
        pos16 = jnp.where(lanes == j, p, pos16)
        return sums16, pos16

    pltpu.async_copy(L.at[n0], rowA, semA)

    def pipe(g, carry):
        s16, p16 = carry
        j0 = 2 * g
        pltpu.make_async_copy(L.at[0], rowA, semA).wait()
        pltpu.async_copy(L.at[n0 + j0 + 1], rowB, semB)
        s16, p16 = gather_one(rowA, j0, s16, p16)
        pltpu.make_async_copy(L.at[0], rowB, semB).wait()

        @pl.when(g < _RPT // 2 - 1)
        def _():
            pltpu.async_copy(L.at[n0 + j0 + 2], rowA, semA)

        s16, p16 = gather_one(rowB, j0 + 1, s16, p16)
        return (s16, p16)

    z16 = jnp.zeros((16,), jnp.float32)
    sums16, pos16 = lax.fori_loop(0, _RPT // 2, pipe, (z16, z16))
    sums_v[...] = sums16
    pos_v[...] = pos16
    pltpu.sync_copy(sums_v, sums_out.at[pl.ds(wid * 16, 16)])
    pltpu.sync_copy(pos_v, pos_out.at[pl.ds(wid * 16, 16)])


def _run_k2(L, bank, qn, idx2, neg2, row_base):
    mesh = plsc.VectorSubcoreMesh(core_axis_name="c", subcore_axis_name="s")
    f = functools.partial(
        pl.kernel,
        out_type=[
            jax.ShapeDtypeStruct((_BSG, _D), jnp.float32),
            jax.ShapeDtypeStruct((512,), jnp.float32),
            jax.ShapeDtypeStruct((512,), jnp.float32),
        ],
        mesh=mesh,
        compiler_params=pltpu.CompilerParams(needs_layout_passes=False),
        scratch_types=[
            pltpu.VMEM((_PACKW,), jnp.int32),
            pltpu.VMEM((_PACKW,), jnp.int32),
            pltpu.VMEM((_K,), jnp.int32),
            pltpu.VMEM((_RPT, _D), jnp.float32),
            pltpu.VMEM((_RPT, _D), jnp.float32),
            pltpu.VMEM((_RPT, _D), jnp.float32),
            pltpu.VMEM((_RPT,), jnp.int32),
            pltpu.VMEM((16,), jnp.float32),
            pltpu.VMEM((16,), jnp.float32),
            pltpu.SemaphoreType.DMA,
            pltpu.SemaphoreType.DMA,
            pltpu.SemaphoreType.DMA,
        ],
    )(functools.partial(_k2_body, row_base))
    return f(L, bank, qn, idx2, neg2)


def _k3_body(copy_ref, copyc_ref, updg_ref, updc_ref, sums_ref, pos_ref,
             idx_ref, out_ref, outc_ref, loss_ref, sem):
    s = sums_ref[...]
    p = pos_ref[...]
    val = jnp.log(s) - p * (1.0 / _TEMP)
    loss_ref[0] = jnp.sum(val[0:1, :]) * (_LAMBDA / _BSG)
    loss_ref[1] = jnp.sum(val[1:2, :]) * (_LAMBDA / _BSG)

    def fire_g(n, _):
        pltpu.make_async_copy(
            updg_ref.at[pl.ds(n, 1)],
            out_ref.at[pl.ds(idx_ref[n], 1)], sem).start()
        return 0

    def fire_c(n, _):
        pltpu.make_async_copy(
            updc_ref.at[pl.ds(n, 1)],
            outc_ref.at[pl.ds(idx_ref[n], 1)], sem).start()
        return 0

    def drain(n, _):
        pltpu.make_async_copy(
            updg_ref.at[pl.ds(0, 1)], out_ref.at[pl.ds(0, 1)], sem).wait()
        return 0

    lax.fori_loop(0, _BSG, fire_g, 0)
    lax.fori_loop(0, _BSG, fire_c, 0)
    lax.fori_loop(0, 2 * _BSG, drain, 0)


def _run_k3(copy, copyc, updg, updc, sums2, pos2, idx):
    return pl.pallas_call(
        _k3_body,
        in_specs=[
            pl.BlockSpec(memory_space=pltpu.MemorySpace.HBM),
            pl.BlockSpec(memory_space=pltpu.MemorySpace.HBM),
            pl.BlockSpec((_BSG, _D), lambda: (0, 0)),
            pl.BlockSpec((_BSG, _D), lambda: (0, 0)),
            pl.BlockSpec((2, _BSG), lambda: (0, 0)),
            pl.BlockSpec((2, _BSG), lambda: (0, 0)),
            pl.BlockSpec(memory_space=pltpu.MemorySpace.SMEM),
        ],
        out_specs=[
            pl.BlockSpec(memory_space=pltpu.MemorySpace.HBM),
            pl.BlockSpec(memory_space=pltpu.MemorySpace.HBM),
            pl.BlockSpec(memory_space=pltpu.MemorySpace.SMEM),
        ],
        out_shape=[
            jax.ShapeDtypeStruct((_M, _D), jnp.float32),
            jax.ShapeDtypeStruct((_M, _D), jnp.float32),
            jax.ShapeDtypeStruct((2,), jnp.float32),
        ],
        input_output_aliases={0: 0, 1: 1},
        scratch_shapes=[pltpu.SemaphoreType.DMA],
    )(copy, copyc, updg, updc, sums2, pos2, idx)


def kernel(q, feature_bank, feature_bank_c, idx, neg_idx):
    neg2 = neg_idx.reshape(_BSG, _K)
    idx2 = idx.reshape(32, _RPT)
    qn, copy, lg = _run_k1(q, feature_bank, 0)
    updg, sumsg, posg = _run_k2(lg, feature_bank, qn, idx2, neg2, 0)
    qnc, copyc, lc = _run_k1(q, feature_bank_c, _BSG)
    updc, sumsc, posc = _run_k2(lc, feature_bank_c, qn, idx2, neg2, _BSG)
    def _gather_lanes(x):
        return x.reshape(32, 16)[:, :_RPT].reshape(_BSG)

    sums2 = jnp.stack([_gather_lanes(sumsg), _gather_lanes(sumsc)])
    pos2 = jnp.stack([_gather_lanes(posg), _gather_lanes(posc)])
    new_bank, new_bank_c, losses = _run_k3(
        copy, copyc, updg, updc, sums2, pos2, idx)
    return (losses, new_bank, new_bank_c)


# speedup vs baseline: 1.1105x; 1.1105x over previous
import functools

import jax
import jax.numpy as jnp
from jax import lax
from jax.experimental import pallas as pl
from jax.experimental.pallas import tpu as pltpu
from jax.experimental.pallas import tpu_sc as plsc

_K = 4096
_LAMBDA = 0.5
_MOM = 0.5
_TEMP = 0.07
_M = 100000
_D = 256
_BS = 256
_BSG = 128

_CHUNK = 2048
_GRID = 49
_MPAD = _CHUNK * _GRID
_HALF = _CHUNK // 2
_PACKW = _MPAD // 2


def _rne_bf16(v):
    u = lax.bitcast_convert_type(v, jnp.int32)
    lsb = lax.shift_right_logical(u, 16) & 1
    return lax.shift_right_logical(u + jnp.int32(0x7FFF) + lsb, 16)


def _pack_bf16_pairs(x):
    lo = _rne_bf16(x[:, :_HALF])
    hi = _rne_bf16(x[:, _HALF:])
    return (hi << 16) | lo


def _k1_body(q_ref, bank_ref, bankc_ref, qn_ref, copy_ref, copyc_ref,
             lg_ref, lc_ref):
    qf = q_ref[...]
    norm = jnp.sqrt(jnp.sum(qf * qf, axis=1, keepdims=True))
    qn = qf / (norm + 1e-12)

    @pl.when(pl.program_id(0) == 0)
    def _():
        qn_ref[...] = qn

    blk = bank_ref[...]
    blkc = bankc_ref[...]
    copy_ref[...] = blk
    copyc_ref[...] = blkc
    qs = qn[:_BSG]
    ql = qn[_BSG:]
    lg = lax.dot_general(qs, blk, (((1,), (1,)), ((), ())),
                         preferred_element_type=jnp.float32)
    lc = lax.dot_general(ql, blkc, (((1,), (1,)), ((), ())),
                         preferred_element_type=jnp.float32)
    lg_ref[...] = _pack_bf16_pairs(lg)
    lc_ref[...] = _pack_bf16_pairs(lc)


def _run_k1(q, bank, bankc):
    return pl.pallas_call(
        _k1_body,
        grid=(_GRID,),
        in_specs=[
            pl.BlockSpec((_BS, _D), lambda i: (0, 0)),
            pl.BlockSpec((_CHUNK, _D), lambda i: (i, 0)),
            pl.BlockSpec((_CHUNK, _D), lambda i: (i, 0)),
        ],
        out_specs=[
            pl.BlockSpec((_BS, _D), lambda i: (0, 0)),
            pl.BlockSpec((_CHUNK, _D), lambda i: (i, 0)),
            pl.BlockSpec((_CHUNK, _D), lambda i: (i, 0)),
            pl.BlockSpec((_BSG, _HALF), lambda i: (0, i)),
            pl.BlockSpec((_BSG, _HALF), lambda i: (0, i)),
        ],
        out_shape=[
            jax.ShapeDtypeStruct((_BS, _D), jnp.float32),
            jax.ShapeDtypeStruct((_M, _D), jnp.float32),
            jax.ShapeDtypeStruct((_M, _D), jnp.float32),
            jax.ShapeDtypeStruct((_BSG, _PACKW), jnp.int32),
            jax.ShapeDtypeStruct((_BSG, _PACKW), jnp.int32),
        ],
    )(q, bank, bankc)


_RPT = 8


def _newton_rsqrt(s):
    i = lax.bitcast_convert_type(s, jnp.int32)
    i = jnp.int32(0x5F3759DF) - lax.shift_right_arithmetic(i, 1)
    y = lax.bitcast_convert_type(i, jnp.float32)
    for _ in range(3):
        y = y * (1.5 - 0.5 * s * y * y)
    return y


def _decode_bf16(buf, g, inv_t):
    w = (lax.shift_right_logical(g, 1) & jnp.int32(-_HALF)) | (g & (_HALF - 1))
    word = plsc.load_gather(buf, [w])
    sh = lax.shift_right_logical(g, 6) & 16
    h = lax.shift_right_logical(word, sh) & jnp.int32(0xFFFF)
    f = plsc.bitcast(lax.shift_left(h, 16), jnp.float32)
    return f, jnp.exp(f * inv_t)


def _sc_rows(L, bank, qn, idx, neg2, upd_out, sums_out, pos_out,
             rowA, rowB, nidx_v, q_v, old_v, upd_v, idx_v, sums_v, pos_v,
             semA, semB, semC, r0, n0):
    inv_t = jnp.float32(1.0 / _TEMP)
    pltpu.sync_copy(idx.at[pl.ds(n0, _RPT)], idx_v)
    pltpu.sync_copy(qn.at[pl.ds(r0, _RPT)], q_v)
    pltpu.async_copy(bank.at[idx_v], old_v, semC).wait()
    lanes = lax.iota(jnp.int32, 16)

    def upd_body(j, _):
        ss = jnp.zeros((16,), jnp.float32)
        for c in range(_D // 16):
            u = (old_v[j, pl.ds(c * 16, 16)] * _MOM
                 + q_v[j, pl.ds(c * 16, 16)] * (1.0 - _MOM))
            ss = ss + u * u
        s2 = jnp.sum(ss)
        t = s2 * _newton_rsqrt(jnp.maximum(s2, jnp.float32(1e-30)))
        r = _newton_rsqrt(t + jnp.float32(1e-12))
        scale = r * r
        for c in range(_D // 16):
            u = (old_v[j, pl.ds(c * 16, 16)] * _MOM
                 + q_v[j, pl.ds(c * 16, 16)] * (1.0 - _MOM))
            upd_v[j, pl.ds(c * 16, 16)] = u * scale
        return 0

    lax.fori_loop(0, _RPT, upd_body, 0)
    pltpu.sync_copy(upd_v, upd_out.at[pl.ds(r0, _RPT)])

    def gather_one(buf, j, sums16, pos16):
        n = n0 + j
        pltpu.sync_copy(neg2.at[n], nidx_v)

        def body(k, accs):
            a0, a1, a2, a3 = accs
            i0 = nidx_v[pl.ds(k * 64, 16)]
            i1 = nidx_v[pl.ds(k * 64 + 16, 16)]
            i2 = nidx_v[pl.ds(k * 64 + 32, 16)]
            i3 = nidx_v[pl.ds(k * 64 + 48, 16)]
            a0 = a0 + _decode_bf16(buf, i0, inv_t)[1]
            a1 = a1 + _decode_bf16(buf, i1, inv_t)[1]
            a2 = a2 + _decode_bf16(buf, i2, inv_t)[1]
            a3 = a3 + _decode_bf16(buf, i3, inv_t)[1]
            return (a0, a1, a2, a3)

        z16 = jnp.zeros((16,), jnp.float32)
        a0, a1, a2, a3 = lax.fori_loop(0, _K // 64, body,
                                       (z16, z16, z16, z16))
        s_neg = jnp.sum((a0 + a1) + (a2 + a3))
        pidx = plsc.load_gather(idx_v, [jnp.full((16,), 0, jnp.int32) + j])
        pvec, pexp = _decode_bf16(buf, pidx, inv_t)
        p = jnp.sum(pvec) * jnp.float32(1.0 / 16.0)
        e_pos = jnp.sum(pexp) * jnp.float32(1.0 / 16.0)
        sums16 = jnp.where(lanes == j, s_neg + e_pos, sums16)
        pos16 = jnp.where(lanes == j, p, pos16)
        return sums16, pos16

    pltpu.async_copy(L.at[n0], rowA, semA)

    def pipe(g, carry):
        s16, p16 = carry
        j0 = 2 * g
        pltpu.make_async_copy(L.at[0], rowA, semA).wait()
        pltpu.async_copy(L.at[n0 + j0 + 1], rowB, semB)
        s16, p16 = gather_one(rowA, j0, s16, p16)
        pltpu.make_async_copy(L.at[0], rowB, semB).wait()

        @pl.when(g < _RPT // 2 - 1)
        def _():
            pltpu.async_copy(L.at[n0 + j0 + 2], rowA, semA)

        s16, p16 = gather_one(rowB, j0 + 1, s16, p16)
        return (s16, p16)

    z16 = jnp.zeros((16,), jnp.float32)
    sums16, pos16 = lax.fori_loop(0, _RPT // 2, pipe, (z16, z16))
    sums_v[...] = sums16
    pos_v[...] = pos16
    pltpu.sync_copy(sums_v.at[pl.ds(0, _RPT)], sums_out.at[pl.ds(r0, _RPT)])
    pltpu.sync_copy(pos_v.at[pl.ds(0, _RPT)], pos_out.at[pl.ds(r0, _RPT)])


def _k2_body(lg, lc, qn, bank, bankc, idx, neg2,
             upd_out, sums_out, pos_out,
             rowA, rowB, nidx_v, q_v, old_v, upd_v, idx_v, sums_v, pos_v,
             semA, semB, semC):
    wid = lax.axis_index("s") * 2 + lax.axis_index("c")
    r0 = wid * _RPT
    n0 = lax.rem(r0, _BSG)
    args = (upd_out, sums_out, pos_out,
            rowA, rowB, nidx_v, q_v, old_v, upd_v, idx_v, sums_v, pos_v,
            semA, semB, semC)

    @pl.when(wid < 16)
    def _():
        _sc_rows(lg, bank, qn, idx, neg2, *args, r0, n0)

    @pl.when(wid >= 16)
    def _():
        _sc_rows(lc, bankc, qn, idx, neg2, *args, r0, n0)


def _run_k2(lg, lc, qn, bank, bankc, idx, neg2):
    mesh = plsc.VectorSubcoreMesh(core_axis_name="c", subcore_axis_name="s")
    f = functools.partial(
        pl.kernel,
        out_type=[
            jax.ShapeDtypeStruct((_BS, _D), jnp.float32),
            jax.ShapeDtypeStruct((_BS,), jnp.float32),
            jax.ShapeDtypeStruct((_BS,), jnp.float32),
        ],
        mesh=mesh,
        compiler_params=pltpu.CompilerParams(needs_layout_passes=False),
        scratch_types=[
            pltpu.VMEM((_PACKW,), jnp.int32),
            pltpu.VMEM((_PACKW,), jnp.int32),
            pltpu.VMEM((_K,), jnp.int32),
            pltpu.VMEM((_RPT, _D), jnp.float32),
            pltpu.VMEM((_RPT, _D), jnp.float32),
            pltpu.VMEM((_RPT, _D), jnp.float32),
            pltpu.VMEM((_RPT,), jnp.int32),
            pltpu.VMEM((16,), jnp.float32),
            pltpu.VMEM((16,), jnp.float32),
            pltpu.SemaphoreType.DMA,
            pltpu.SemaphoreType.DMA,
            pltpu.SemaphoreType.DMA,
        ],
    )(_k2_body)
    return f(lg, lc, qn, bank, bankc, idx, neg2)


def _k3_body(copy_ref, copyc_ref, upd_ref, sums_ref, pos_ref, idx_ref,
             out_ref, outc_ref, loss_ref, sem):
    s = sums_ref[...]
    p = pos_ref[...]
    val = jnp.log(s) - p * (1.0 / _TEMP)
    loss_ref[0] = jnp.sum(val[0:1, :]) * (_LAMBDA / _BSG)
    loss_ref[1] = jnp.sum(val[1:2, :]) * (_LAMBDA / _BSG)

    def fire_g(n, _):
        pltpu.make_async_copy(
            upd_ref.at[pl.ds(n, 1)],
            out_ref.at[pl.ds(idx_ref[n], 1)], sem).start()
        return 0

    def fire_c(n, _):
        pltpu.make_async_copy(
            upd_ref.at[pl.ds(_BSG + n, 1)],
            outc_ref.at[pl.ds(idx_ref[n], 1)], sem).start()
        return 0

    def drain(n, _):
        pltpu.make_async_copy(
            upd_ref.at[pl.ds(0, 1)], out_ref.at[pl.ds(0, 1)], sem).wait()
        return 0

    lax.fori_loop(0, _BSG, fire_g, 0)
    lax.fori_loop(0, _BSG, fire_c, 0)
    lax.fori_loop(0, 2 * _BSG, drain, 0)


def _run_k3(copy, copyc, upd, sums2, pos2, idx):
    return pl.pallas_call(
        _k3_body,
        in_specs=[
            pl.BlockSpec(memory_space=pltpu.MemorySpace.HBM),
            pl.BlockSpec(memory_space=pltpu.MemorySpace.HBM),
            pl.BlockSpec((_BS, _D), lambda: (0, 0)),
            pl.BlockSpec((2, _BSG), lambda: (0, 0)),
            pl.BlockSpec((2, _BSG), lambda: (0, 0)),
            pl.BlockSpec(memory_space=pltpu.SMEM),
        ],
        out_specs=[
            pl.BlockSpec(memory_space=pltpu.MemorySpace.HBM),
            pl.BlockSpec(memory_space=pltpu.MemorySpace.HBM),
            pl.BlockSpec(memory_space=pltpu.SMEM),
        ],
        out_shape=[
            jax.ShapeDtypeStruct((_M, _D), jnp.float32),
            jax.ShapeDtypeStruct((_M, _D), jnp.float32),
            jax.ShapeDtypeStruct((2,), jnp.float32),
        ],
        input_output_aliases={0: 0, 1: 1},
        scratch_shapes=[pltpu.SemaphoreType.DMA],
    )(copy, copyc, upd, sums2, pos2, idx)


def kernel(q, feature_bank, feature_bank_c, idx, neg_idx):
    neg2 = neg_idx.reshape(_BSG, _K)
    qn, copy, copyc, lg, lc = _run_k1(q, feature_bank, feature_bank_c)
    upd, sums, pos = _run_k2(lg, lc, qn, feature_bank, feature_bank_c,
                             idx, neg2)
    new_bank, new_bank_c, losses = _run_k3(
        copy, copyc, upd, sums.reshape(2, _BSG), pos.reshape(2, _BSG), idx)
    return (losses, new_bank, new_bank_c)


# speedup vs baseline: 1.1345x; 1.0217x over previous
import functools

import jax
import jax.numpy as jnp
from jax import lax
from jax.experimental import pallas as pl
from jax.experimental.pallas import tpu as pltpu
from jax.experimental.pallas import tpu_sc as plsc

_K = 4096
_LAMBDA = 0.5
_MOM = 0.5
_TEMP = 0.07
_M = 100000
_D = 256
_BS = 256
_BSG = 128

_CHUNK = 4096
_GRID = 25
_MPAD = _CHUNK * _GRID
_HALF = _CHUNK // 2
_PACKW = _MPAD // 2


def _rne_bf16(v):
    u = lax.bitcast_convert_type(v, jnp.int32)
    lsb = lax.shift_right_logical(u, 16) & 1
    return lax.shift_right_logical(u + jnp.int32(0x7FFF) + lsb, 16)


def _pack_bf16_pairs(x):
    lo = _rne_bf16(x[:, :_HALF])
    hi = _rne_bf16(x[:, _HALF:])
    return (hi << 16) | lo


def _k1_body(q_ref, bank_ref, bankc_ref, qn_ref, copy_ref, copyc_ref,
             lg_ref, lc_ref):
    qf = q_ref[...]
    norm = jnp.sqrt(jnp.sum(qf * qf, axis=1, keepdims=True))
    qn = qf / (norm + 1e-12)

    @pl.when(pl.program_id(0) == 0)
    def _():
        qn_ref[...] = qn

    blk = bank_ref[...]
    blkc = bankc_ref[...]
    copy_ref[...] = blk
    copyc_ref[...] = blkc
    qs = qn[:_BSG]
    ql = qn[_BSG:]
    lg = lax.dot_general(qs, blk, (((1,), (1,)), ((), ())),
                         preferred_element_type=jnp.float32)
    lc = lax.dot_general(ql, blkc, (((1,), (1,)), ((), ())),
                         preferred_element_type=jnp.float32)
    lg_ref[...] = _pack_bf16_pairs(lg)
    lc_ref[...] = _pack_bf16_pairs(lc)


def _run_k1(q, bank, bankc):
    return pl.pallas_call(
        _k1_body,
        grid=(_GRID,),
        in_specs=[
            pl.BlockSpec((_BS, _D), lambda i: (0, 0)),
            pl.BlockSpec((_CHUNK, _D), lambda i: (i, 0)),
            pl.BlockSpec((_CHUNK, _D), lambda i: (i, 0)),
        ],
        out_specs=[
            pl.BlockSpec((_BS, _D), lambda i: (0, 0)),
            pl.BlockSpec((_CHUNK, _D), lambda i: (i, 0)),
            pl.BlockSpec((_CHUNK, _D), lambda i: (i, 0)),
            pl.BlockSpec((_BSG, _HALF), lambda i: (0, i)),
            pl.BlockSpec((_BSG, _HALF), lambda i: (0, i)),
        ],
        out_shape=[
            jax.ShapeDtypeStruct((_BS, _D), jnp.float32),
            jax.ShapeDtypeStruct((_M, _D), jnp.float32),
            jax.ShapeDtypeStruct((_M, _D), jnp.float32),
            jax.ShapeDtypeStruct((_BSG, _PACKW), jnp.int32),
            jax.ShapeDtypeStruct((_BSG, _PACKW), jnp.int32),
        ],
    )(q, bank, bankc)


_RPT = 8


def _newton_rsqrt(s):
    i = lax.bitcast_convert_type(s, jnp.int32)
    i = jnp.int32(0x5F3759DF) - lax.shift_right_arithmetic(i, 1)
    y = lax.bitcast_convert_type(i, jnp.float32)
    for _ in range(3):
        y = y * (1.5 - 0.5 * s * y * y)
    return y


def _decode_bf16(buf, g, inv_t):
    w = (lax.shift_right_logical(g, 1) & jnp.int32(-_HALF)) | (g & (_HALF - 1))
    word = plsc.load_gather(buf, [w])
    sh = lax.shift_right_logical(g, _HALF.bit_length() - 5) & 16
    h = lax.shift_right_logical(word, sh) & jnp.int32(0xFFFF)
    f = plsc.bitcast(lax.shift_left(h, 16), jnp.float32)
    return f, jnp.exp(f * inv_t)


def _sc_rows(L, bank, qn, idx, neg2, upd_out, sums_out, pos_out,
             rowA, rowB, nidx_v, q_v, old_v, upd_v, idx_v, sums_v, pos_v,
             semA, semB, semC, r0, n0):
    inv_t = jnp.float32(1.0 / _TEMP)
    pltpu.sync_copy(idx.at[pl.ds(n0, _RPT)], idx_v)
    pltpu.sync_copy(qn.at[pl.ds(r0, _RPT)], q_v)
    pltpu.async_copy(bank.at[idx_v], old_v, semC).wait()
    lanes = lax.iota(jnp.int32, 16)

    def upd_body(j, _):
        ss = jnp.zeros((16,), jnp.float32)
        for c in range(_D // 16):
            u = (old_v[j, pl.ds(c * 16, 16)] * _MOM
                 + q_v[j, pl.ds(c * 16, 16)] * (1.0 - _MOM))
            ss = ss + u * u
        s2 = jnp.sum(ss)
        t = s2 * _newton_rsqrt(jnp.maximum(s2, jnp.float32(1e-30)))
        r = _newton_rsqrt(t + jnp.float32(1e-12))
        scale = r * r
        for c in range(_D // 16):
            u = (old_v[j, pl.ds(c * 16, 16)] * _MOM
                 + q_v[j, pl.ds(c * 16, 16)] * (1.0 - _MOM))
            upd_v[j, pl.ds(c * 16, 16)] = u * scale
        return 0

    lax.fori_loop(0, _RPT, upd_body, 0)
    pltpu.sync_copy(upd_v, upd_out.at[pl.ds(r0, _RPT)])

    def gather_one(buf, j, sums16, pos16):
        n = n0 + j
        pltpu.sync_copy(neg2.at[n], nidx_v)

        def body(k, accs):
            a0, a1, a2, a3 = accs
            i0 = nidx_v[pl.ds(k * 64, 16)]
            i1 = nidx_v[pl.ds(k * 64 + 16, 16)]
            i2 = nidx_v[pl.ds(k * 64 + 32, 16)]
            i3 = nidx_v[pl.ds(k * 64 + 48, 16)]
            a0 = a0 + _decode_bf16(buf, i0, inv_t)[1]
            a1 = a1 + _decode_bf16(buf, i1, inv_t)[1]
            a2 = a2 + _decode_bf16(buf, i2, inv_t)[1]
            a3 = a3 + _decode_bf16(buf, i3, inv_t)[1]
            return (a0, a1, a2, a3)

        z16 = jnp.zeros((16,), jnp.float32)
        a0, a1, a2, a3 = lax.fori_loop(0, _K // 64, body,
                                       (z16, z16, z16, z16))
        s_neg = jnp.sum((a0 + a1) + (a2 + a3))
        pidx = plsc.load_gather(idx_v, [jnp.full((16,), 0, jnp.int32) + j])
        pvec, pexp = _decode_bf16(buf, pidx, inv_t)
        p = jnp.sum(pvec) * jnp.float32(1.0 / 16.0)
        e_pos = jnp.sum(pexp) * jnp.float32(1.0 / 16.0)
        sums16 = jnp.where(lanes == j, s_neg + e_pos, sums16)
        pos16 = jnp.where(lanes == j, p, pos16)
        return sums16, pos16

    pltpu.async_copy(L.at[n0], rowA, semA)

    def pipe(g, carry):
        s16, p16 = carry
        j0 = 2 * g
        pltpu.make_async_copy(L.at[0], rowA, semA).wait()
        pltpu.async_copy(L.at[n0 + j0 + 1], rowB, semB)
        s16, p16 = gather_one(rowA, j0, s16, p16)
        pltpu.make_async_copy(L.at[0], rowB, semB).wait()

        @pl.when(g < _RPT // 2 - 1)
        def _():
            pltpu.async_copy(L.at[n0 + j0 + 2], rowA, semA)

        s16, p16 = gather_one(rowB, j0 + 1, s16, p16)
        return (s16, p16)

    z16 = jnp.zeros((16,), jnp.float32)
    sums16, pos16 = lax.fori_loop(0, _RPT // 2, pipe, (z16, z16))
    sums_v[...] = sums16
    pos_v[...] = pos16
    pltpu.sync_copy(sums_v.at[pl.ds(0, _RPT)], sums_out.at[pl.ds(r0, _RPT)])
    pltpu.sync_copy(pos_v.at[pl.ds(0, _RPT)], pos_out.at[pl.ds(r0, _RPT)])


def _k2_body(lg, lc, qn, bank, bankc, idx, neg2,
             upd_out, sums_out, pos_out,
             rowA, rowB, nidx_v, q_v, old_v, upd_v, idx_v, sums_v, pos_v,
             semA, semB, semC):
    wid = lax.axis_index("s") * 2 + lax.axis_index("c")
    r0 = wid * _RPT
    n0 = lax.rem(r0, _BSG)
    args = (upd_out, sums_out, pos_out,
            rowA, rowB, nidx_v, q_v, old_v, upd_v, idx_v, sums_v, pos_v,
            semA, semB, semC)

    @pl.when(wid < 16)
    def _():
        _sc_rows(lg, bank, qn, idx, neg2, *args, r0, n0)

    @pl.when(wid >= 16)
    def _():
        _sc_rows(lc, bankc, qn, idx, neg2, *args, r0, n0)


def _run_k2(lg, lc, qn, bank, bankc, idx, neg2):
    mesh = plsc.VectorSubcoreMesh(core_axis_name="c", subcore_axis_name="s")
    f = functools.partial(
        pl.kernel,
        out_type=[
            jax.ShapeDtypeStruct((_BS, _D), jnp.float32),
            jax.ShapeDtypeStruct((_BS,), jnp.float32),
            jax.ShapeDtypeStruct((_BS,), jnp.float32),
        ],
        mesh=mesh,
        compiler_params=pltpu.CompilerParams(needs_layout_passes=False),
        scratch_types=[
            pltpu.VMEM((_PACKW,), jnp.int32),
            pltpu.VMEM((_PACKW,), jnp.int32),
            pltpu.VMEM((_K,), jnp.int32),
            pltpu.VMEM((_RPT, _D), jnp.float32),
            pltpu.VMEM((_RPT, _D), jnp.float32),
            pltpu.VMEM((_RPT, _D), jnp.float32),
            pltpu.VMEM((_RPT,), jnp.int32),
            pltpu.VMEM((16,), jnp.float32),
            pltpu.VMEM((16,), jnp.float32),
            pltpu.SemaphoreType.DMA,
            pltpu.SemaphoreType.DMA,
            pltpu.SemaphoreType.DMA,
        ],
    )(_k2_body)
    return f(lg, lc, qn, bank, bankc, idx, neg2)


def _k3_body(copy_ref, copyc_ref, upd_ref, sums_ref, pos_ref, idx_ref,
             out_ref, outc_ref, loss_ref, sem):
    s = sums_ref[...]
    p = pos_ref[...]
    val = jnp.log(s) - p * (1.0 / _TEMP)
    loss_ref[0] = jnp.sum(val[0:1, :]) * (_LAMBDA / _BSG)
    loss_ref[1] = jnp.sum(val[1:2, :]) * (_LAMBDA / _BSG)

    def fire_g(n, _):
        pltpu.make_async_copy(
            upd_ref.at[pl.ds(n, 1)],
            out_ref.at[pl.ds(idx_ref[n], 1)], sem).start()
        return 0

    def fire_c(n, _):
        pltpu.make_async_copy(
            upd_ref.at[pl.ds(_BSG + n, 1)],
            outc_ref.at[pl.ds(idx_ref[n], 1)], sem).start()
        return 0

    def drain(n, _):
        pltpu.make_async_copy(
            upd_ref.at[pl.ds(0, 1)], out_ref.at[pl.ds(0, 1)], sem).wait()
        return 0

    lax.fori_loop(0, _BSG, fire_g, 0)
    lax.fori_loop(0, _BSG, fire_c, 0)
    lax.fori_loop(0, 2 * _BSG, drain, 0)


def _run_k3(copy, copyc, upd, sums2, pos2, idx):
    return pl.pallas_call(
        _k3_body,
        in_specs=[
            pl.BlockSpec(memory_space=pltpu.MemorySpace.HBM),
            pl.BlockSpec(memory_space=pltpu.MemorySpace.HBM),
            pl.BlockSpec((_BS, _D), lambda: (0, 0)),
            pl.BlockSpec((2, _BSG), lambda: (0, 0)),
            pl.BlockSpec((2, _BSG), lambda: (0, 0)),
            pl.BlockSpec(memory_space=pltpu.SMEM),
        ],
        out_specs=[
            pl.BlockSpec(memory_space=pltpu.MemorySpace.HBM),
            pl.BlockSpec(memory_space=pltpu.MemorySpace.HBM),
            pl.BlockSpec(memory_space=pltpu.SMEM),
        ],
        out_shape=[
            jax.ShapeDtypeStruct((_M, _D), jnp.float32),
            jax.ShapeDtypeStruct((_M, _D), jnp.float32),
            jax.ShapeDtypeStruct((2,), jnp.float32),
        ],
        input_output_aliases={0: 0, 1: 1},
        scratch_shapes=[pltpu.SemaphoreType.DMA],
    )(copy, copyc, upd, sums2, pos2, idx)


def kernel(q, feature_bank, feature_bank_c, idx, neg_idx):
    neg2 = neg_idx.reshape(_BSG, _K)
    qn, copy, copyc, lg, lc = _run_k1(q, feature_bank, feature_bank_c)
    upd, sums, pos = _run_k2(lg, lc, qn, feature_bank, feature_bank_c,
                             idx, neg2)
    new_bank, new_bank_c, losses = _run_k3(
        copy, copyc, upd, sums.reshape(2, _BSG), pos.reshape(2, _BSG), idx)
    return (losses, new_bank, new_bank_c)
